# baseline (speedup 1.0000x reference)
import functools

import jax
import jax.numpy as jnp
from jax import lax
from jax.experimental import pallas as pl
from jax.experimental.pallas import tpu as pltpu
from jax.experimental.pallas import tpu_sc as plsc

D = 128
K = 4
GW = K * D // 2
NC, NS = 2, 16
NW = NC * NS
CH = 56
NCH0, NCH1 = 36, 20


def _mm_body(x_ref, w0_ref, wn_ref, b_ref, g_ref, o_ref):
    g = g_ref[...]
    g_lo = lax.bitcast_convert_type(g << 16, jnp.float32)
    g_hi = lax.bitcast_convert_type(g & jnp.int32(-65536), jnp.float32)
    xg = jnp.concatenate([g_lo, g_hi], axis=1)
    o_ref[...] = (
        jnp.dot(x_ref[...], w0_ref[...], preferred_element_type=jnp.float32)
        + b_ref[...]
        + jnp.dot(xg, wn_ref[...], preferred_element_type=jnp.float32)
    )


def _make_sc_kernel(n_rows: int, np_rows: int):
    assert np_rows == NS * (NCH0 + NCH1) * CH
    assert NCH0 % 2 == 0 and NCH1 % 2 == 0
    mesh = plsc.VectorSubcoreMesh(
        core_axis_name="c", subcore_axis_name="s", num_cores=NC, num_subcores=NS
    )

    @functools.partial(
        pl.kernel,
        out_type=jax.ShapeDtypeStruct((np_rows, GW), jnp.int32),
        mesh=mesh,
        scratch_types=[
            pltpu.VMEM((max(NCH0, NCH1), K // 2, 2 * CH), jnp.int32),
            pltpu.VMEM((2, K // 2, 2 * CH, D), jnp.float32),
            pltpu.VMEM((2, CH, GW), jnp.int32),
            pltpu.SemaphoreType.DMA,
            pltpu.SemaphoreType.DMA,
            pltpu.SemaphoreType.DMA,
            pltpu.SemaphoreType.DMA,
        ],
        compiler_params=pltpu.CompilerParams(needs_layout_passes=False),
    )
    def sc_gather_pack(x_hbm, idxf_hbm, g_hbm,
                       idx_s, rows_s, gp_s, sem_g0, sem_g1, sem_o0, sem_o1):
        sid = lax.axis_index("s")
        cid = lax.axis_index("c")
        g0 = sid * (NCH0 + NCH1) + cid * NCH0
        nch_w = jnp.where(cid == 1, NCH1, NCH0)
        wbase = g0 * CH
        sem_g = (sem_g0, sem_g1)
        sem_o = (sem_o0, sem_o1)

        def in_copies(c, p):
            return [
                pltpu.make_async_copy(
                    x_hbm.at[idx_s.at[c, kk]], rows_s.at[p, kk], sem_g[p]
                )
                for kk in range(K // 2)
            ]

        def out_copy(c, p):
            base = wbase + c * CH
            return pltpu.make_async_copy(
                gp_s.at[p], g_hbm.at[pl.ds(base, CH)], sem_o[p]
            )

        def stage_and_fire(c, p):
            for cp in in_copies(c, p):
                cp.start()

        def chunk_step(c, p):
            q = 1 - p

            @pl.when(c + 1 < nch_w)
            def _fire_next():
                @pl.when(c >= 1)
                def _drain_prev_out():
                    out_copy(c - 1, q).wait()

                stage_and_fire(c + 1, q)

            for cp in in_copies(c, p):
                cp.wait()

            @plsc.parallel_loop(0, CH, unroll=4)
            def row_body(i):
                for k in range(K):
                    kk, off = k // 2, (k % 2) * CH
                    for j in range(D // 32):
                        a = rows_s[p, kk, off + i, pl.ds(j * 32, 16)]
                        bvec = rows_s[p, kk, off + i, pl.ds(j * 32 + 16, 16)]
                        packed = plsc.pack(
                            a, bvec, format=plsc.PackFormat.INTERLEAVED
                        )
                        gp_s[p, i, pl.ds(k * (D // 2) + j * 16, 16)] = (
                            plsc.bitcast(packed, jnp.int32)
                        )
            out_copy(c, p).start()

        @pl.when(cid == 0)
        def _stage_idx0():
            pltpu.sync_copy(
                idxf_hbm.at[pl.ds(g0, NCH0)], idx_s.at[pl.ds(0, NCH0)]
            )

        @pl.when(cid == 1)
        def _stage_idx1():
            pltpu.sync_copy(
                idxf_hbm.at[pl.ds(g0, NCH1)], idx_s.at[pl.ds(0, NCH1)]
            )

        stage_and_fire(0, 0)

        def pair_body(t, carry):
            chunk_step(2 * t, 0)
            chunk_step(2 * t + 1, 1)
            return carry

        lax.fori_loop(0, nch_w // 2, pair_body, 0)

        out_copy(nch_w - 2, 0).wait()
        out_copy(nch_w - 1, 1).wait()

    return sc_gather_pack


def kernel(x, neighbor_idx, W, b):
    n, d = x.shape
    assert d == D
    np_rows = ((n + NW * CH - 1) // (NW * CH)) * (NW * CH)
    np_half = np_rows // 2
    blk = 1792

    idxf = (
        jnp.pad(neighbor_idx.astype(jnp.int32).T, ((0, 0), (0, np_rows - n)))
        .reshape(K, np_rows // CH, CH)
        .transpose(1, 0, 2)
        .reshape(np_rows // CH, K // 2, 2 * CH)
    )

    wt = W.T
    w0 = wt[0:D]
    pos = jnp.arange(2 * GW)
    part, w = pos // GW, pos % GW
    kk, jj, cc = w // (D // 2), (w % (D // 2)) // 16, w % 16
    orig = D * kk + 32 * jj + 16 * part + cc
    wn = wt[D:][orig]
    b2 = b.reshape(1, D)

    sc = _make_sc_kernel(n, np_half)
    half_chunks = np_half // CH
    g_a = sc(x, idxf[:half_chunks])
    g_b = sc(x, idxf[half_chunks:])

    def mm_call(stripe, g_s, rows_out):
        base_blk = stripe * (np_half // blk)
        grid = (rows_out + blk - 1) // blk
        return pl.pallas_call(
            _mm_body,
            grid=(grid,),
            in_specs=[
                pl.BlockSpec((blk, D), lambda i: (base_blk + i, 0)),
                pl.BlockSpec((D, D), lambda i: (0, 0)),
                pl.BlockSpec((2 * GW, D), lambda i: (0, 0)),
                pl.BlockSpec((1, D), lambda i: (0, 0)),
                pl.BlockSpec((blk, GW), lambda i: (i, 0)),
            ],
            out_specs=pl.BlockSpec((blk, D), lambda i: (i, 0)),
            out_shape=jax.ShapeDtypeStruct((rows_out, D), jnp.float32),
        )(x, w0, wn, b2, g_s)

    o_a = mm_call(0, g_a, np_half)
    o_b = mm_call(1, g_b, n - np_half)
    return jnp.concatenate([o_a, o_b], axis=0)


# speedup vs baseline: 1.1375x; 1.1375x over previous
import functools

import jax
import jax.numpy as jnp
from jax import lax
from jax.experimental import pallas as pl
from jax.experimental.pallas import tpu as pltpu
from jax.experimental.pallas import tpu_sc as plsc

D = 128
K = 4
GW = K * D // 2
NC, NS = 2, 16
NW = NC * NS
CH = 64
NCH0, NCH1 = 64, 34


def _mm_body(x_ref, w0_ref, wn_ref, b_ref, g_ref, o_ref):
    g = g_ref[...]
    g_lo = lax.bitcast_convert_type(g << 16, jnp.float32)
    g_hi = lax.bitcast_convert_type(g & jnp.int32(-65536), jnp.float32)
    xg = jnp.concatenate([g_lo, g_hi], axis=1)
    o_ref[...] = (
        jnp.dot(x_ref[...], w0_ref[...], preferred_element_type=jnp.float32)
        + b_ref[...]
        + jnp.dot(xg, wn_ref[...], preferred_element_type=jnp.float32)
    )


def _make_sc_kernel(n_rows: int, np_rows: int):
    assert np_rows == NS * (NCH0 + NCH1) * CH
    assert NCH0 % 2 == 0 and NCH1 % 2 == 0
    mesh = plsc.VectorSubcoreMesh(
        core_axis_name="c", subcore_axis_name="s", num_cores=NC, num_subcores=NS
    )

    @functools.partial(
        pl.kernel,
        out_type=jax.ShapeDtypeStruct((np_rows, GW), jnp.int32),
        mesh=mesh,
        scratch_types=[
            pltpu.VMEM((max(NCH0, NCH1), K // 2, 2 * CH), jnp.int32),
            pltpu.VMEM((2, K // 2, 2 * CH, D), jnp.float32),
            pltpu.VMEM((2, CH, GW), jnp.int32),
            pltpu.SemaphoreType.DMA,
            pltpu.SemaphoreType.DMA,
            pltpu.SemaphoreType.DMA,
            pltpu.SemaphoreType.DMA,
        ],
        compiler_params=pltpu.CompilerParams(needs_layout_passes=False),
    )
    def sc_gather_pack(x_hbm, idxf_hbm, g_hbm,
                       idx_s, rows_s, gp_s, sem_g0, sem_g1, sem_o0, sem_o1):
        sid = lax.axis_index("s")
        cid = lax.axis_index("c")
        g0 = sid * (NCH0 + NCH1) + cid * NCH0
        nch_w = jnp.where(cid == 1, NCH1, NCH0)
        wbase = g0 * CH
        sem_g = (sem_g0, sem_g1)
        sem_o = (sem_o0, sem_o1)

        def in_copies(c, p):
            return [
                pltpu.make_async_copy(
                    x_hbm.at[idx_s.at[c, kk]], rows_s.at[p, kk], sem_g[p]
                )
                for kk in range(K // 2)
            ]

        def out_copy(c, p):
            base = wbase + c * CH
            return pltpu.make_async_copy(
                gp_s.at[p], g_hbm.at[pl.ds(base, CH)], sem_o[p]
            )

        def stage_and_fire(c, p):
            for cp in in_copies(c, p):
                cp.start()

        def chunk_step(c, p):
            q = 1 - p

            @pl.when(c + 1 < nch_w)
            def _fire_next():
                @pl.when(c >= 1)
                def _drain_prev_out():
                    out_copy(c - 1, q).wait()

                stage_and_fire(c + 1, q)

            for cp in in_copies(c, p):
                cp.wait()

            @plsc.parallel_loop(0, CH, unroll=4)
            def row_body(i):
                for k in range(K):
                    kk, off = k // 2, (k % 2) * CH
                    for j in range(D // 32):
                        a = rows_s[p, kk, off + i, pl.ds(j * 32, 16)]
                        bvec = rows_s[p, kk, off + i, pl.ds(j * 32 + 16, 16)]
                        packed = plsc.pack(
                            a, bvec, format=plsc.PackFormat.INTERLEAVED
                        )
                        gp_s[p, i, pl.ds(k * (D // 2) + j * 16, 16)] = (
                            plsc.bitcast(packed, jnp.int32)
                        )
            out_copy(c, p).start()

        @pl.when(cid == 0)
        def _stage_idx0():
            pltpu.sync_copy(
                idxf_hbm.at[pl.ds(g0, NCH0)], idx_s.at[pl.ds(0, NCH0)]
            )

        @pl.when(cid == 1)
        def _stage_idx1():
            pltpu.sync_copy(
                idxf_hbm.at[pl.ds(g0, NCH1)], idx_s.at[pl.ds(0, NCH1)]
            )

        stage_and_fire(0, 0)

        def pair_body(t, carry):
            chunk_step(2 * t, 0)
            chunk_step(2 * t + 1, 1)
            return carry

        lax.fori_loop(0, nch_w // 2, pair_body, 0)

        out_copy(nch_w - 2, 0).wait()
        out_copy(nch_w - 1, 1).wait()

    return sc_gather_pack


def kernel(x, neighbor_idx, W, b):
    n, d = x.shape
    assert d == D
    np_rows = ((n + NW * CH - 1) // (NW * CH)) * (NW * CH)
    blk = 2048

    idxf = (
        jnp.pad(neighbor_idx.astype(jnp.int32).T, ((0, 0), (0, np_rows - n)))
        .reshape(K, np_rows // CH, CH)
        .transpose(1, 0, 2)
        .reshape(np_rows // CH, K // 2, 2 * CH)
    )

    wt = W.T
    w0 = wt[0:D]
    pos = jnp.arange(2 * GW)
    part, w = pos // GW, pos % GW
    kk, jj, cc = w // (D // 2), (w % (D // 2)) // 16, w % 16
    orig = D * kk + 32 * jj + 16 * part + cc
    wn = wt[D:][orig]
    b2 = b.reshape(1, D)

    g = _make_sc_kernel(n, np_rows)(x, idxf)

    n_blocks = (n + blk - 1) // blk
    mm = pl.pallas_call(
        _mm_body,
        grid=(n_blocks,),
        in_specs=[
            pl.BlockSpec((blk, D), lambda i: (i, 0)),
            pl.BlockSpec((D, D), lambda i: (0, 0)),
            pl.BlockSpec((2 * GW, D), lambda i: (0, 0)),
            pl.BlockSpec((1, D), lambda i: (0, 0)),
            pl.BlockSpec((blk, GW), lambda i: (i, 0)),
        ],
        out_specs=pl.BlockSpec((blk, D), lambda i: (i, 0)),
        out_shape=jax.ShapeDtypeStruct((n, D), jnp.float32),
    )
    return mm(x, w0, wn, b2, g)


# speedup vs baseline: 1.1948x; 1.0504x over previous
import functools

import jax
import jax.numpy as jnp
from jax import lax
from jax.experimental import pallas as pl
from jax.experimental.pallas import tpu as pltpu
from jax.experimental.pallas import tpu_sc as plsc

D = 128
K = 4
GW = K * D // 2
NC, NS = 2, 16
NW = NC * NS
CH = 64
NCH0, NCH1 = 64, 34


def _mm_body(x_ref, w0_ref, wn_ref, b_ref, g_ref, o_ref):
    g = g_ref[...]
    g_lo = lax.bitcast_convert_type(g << 16, jnp.float32)
    g_hi = lax.bitcast_convert_type(g & jnp.int32(-65536), jnp.float32)
    xg = jnp.concatenate([g_lo, g_hi], axis=1)
    o_ref[...] = (
        jnp.dot(x_ref[...], w0_ref[...], preferred_element_type=jnp.float32)
        + b_ref[...]
        + jnp.dot(xg, wn_ref[...], preferred_element_type=jnp.float32)
    )


def _make_sc_kernel(n_rows: int, np_rows: int):
    assert np_rows == NS * (NCH0 + NCH1) * CH
    assert NCH0 % 2 == 0 and NCH1 % 2 == 0
    mesh = plsc.VectorSubcoreMesh(
        core_axis_name="c", subcore_axis_name="s", num_cores=NC, num_subcores=NS
    )

    @functools.partial(
        pl.kernel,
        out_type=jax.ShapeDtypeStruct((np_rows, GW), jnp.int32),
        mesh=mesh,
        scratch_types=[
            pltpu.VMEM((max(NCH0, NCH1), K // 2, 2 * CH), jnp.int32),
            pltpu.VMEM((2, K // 2, 2 * CH, D), jnp.float32),
            pltpu.VMEM((2, CH, GW), jnp.int32),
            pltpu.SemaphoreType.DMA,
            pltpu.SemaphoreType.DMA,
            pltpu.SemaphoreType.DMA,
            pltpu.SemaphoreType.DMA,
        ],
        compiler_params=pltpu.CompilerParams(needs_layout_passes=False),
    )
    def sc_gather_pack(x_hbm, idxf_hbm, g_hbm,
                       idx_s, rows_s, gp_s, sem_g0, sem_g1, sem_o0, sem_o1):
        sid = lax.axis_index("s")
        cid = lax.axis_index("c")
        g0 = sid * (NCH0 + NCH1) + cid * NCH0
        nch_w = jnp.where(cid == 1, NCH1, NCH0)
        wbase = g0 * CH
        sem_g = (sem_g0, sem_g1)
        sem_o = (sem_o0, sem_o1)

        def in_copies(c, p):
            return [
                pltpu.make_async_copy(
                    x_hbm.at[idx_s.at[c, kk]], rows_s.at[p, kk], sem_g[p]
                )
                for kk in range(K // 2)
            ]

        def out_copy(c, p):
            base = wbase + c * CH
            return pltpu.make_async_copy(
                gp_s.at[p], g_hbm.at[pl.ds(base, CH)], sem_o[p]
            )

        def stage_and_fire(c, p):
            for cp in in_copies(c, p):
                cp.start()

        def chunk_step(c, p):
            q = 1 - p

            @pl.when(c + 1 < nch_w)
            def _fire_next():
                @pl.when(c >= 1)
                def _drain_prev_out():
                    out_copy(c - 1, q).wait()

                stage_and_fire(c + 1, q)

            for cp in in_copies(c, p):
                cp.wait()

            @plsc.parallel_loop(0, CH, unroll=4)
            def row_body(i):
                for k in range(K):
                    kk, off = k // 2, (k % 2) * CH
                    for j in range(D // 32):
                        a = rows_s[p, kk, off + i, pl.ds(j * 32, 16)]
                        bvec = rows_s[p, kk, off + i, pl.ds(j * 32 + 16, 16)]
                        packed = plsc.pack(
                            a, bvec, format=plsc.PackFormat.INTERLEAVED
                        )
                        gp_s[p, i, pl.ds(k * (D // 2) + j * 16, 16)] = (
                            plsc.bitcast(packed, jnp.int32)
                        )
            out_copy(c, p).start()

        @pl.when(cid == 0)
        def _stage_idx0():
            pltpu.sync_copy(
                idxf_hbm.at[pl.ds(g0, NCH0)], idx_s.at[pl.ds(0, NCH0)]
            )

        @pl.when(cid == 1)
        def _stage_idx1():
            pltpu.sync_copy(
                idxf_hbm.at[pl.ds(g0, NCH1)], idx_s.at[pl.ds(0, NCH1)]
            )

        stage_and_fire(0, 0)

        def pair_body(t, carry):
            chunk_step(2 * t, 0)
            chunk_step(2 * t + 1, 1)
            return carry

        lax.fori_loop(0, nch_w // 2, pair_body, 0)

        out_copy(nch_w - 2, 0).wait()
        out_copy(nch_w - 1, 1).wait()

    return sc_gather_pack


def kernel(x, neighbor_idx, W, b):
    n, d = x.shape
    assert d == D
    np_rows = ((n + NW * CH - 1) // (NW * CH)) * (NW * CH)
    blk = 4096

    idxf = (
        jnp.pad(neighbor_idx.astype(jnp.int32).T, ((0, 0), (0, np_rows - n)))
        .reshape(K, np_rows // CH, CH)
        .transpose(1, 0, 2)
        .reshape(np_rows // CH, K // 2, 2 * CH)
    )

    wt = W.T
    w0 = wt[0:D]
    pos = jnp.arange(2 * GW)
    part, w = pos // GW, pos % GW
    kk, jj, cc = w // (D // 2), (w % (D // 2)) // 16, w % 16
    orig = D * kk + 32 * jj + 16 * part + cc
    wn = wt[D:][orig]
    b2 = b.reshape(1, D)

    g = _make_sc_kernel(n, np_rows)(x, idxf)

    n_blocks = (n + blk - 1) // blk
    mm = pl.pallas_call(
        _mm_body,
        grid=(n_blocks,),
        in_specs=[
            pl.BlockSpec((blk, D), lambda i: (i, 0)),
            pl.BlockSpec((D, D), lambda i: (0, 0)),
            pl.BlockSpec((2 * GW, D), lambda i: (0, 0)),
            pl.BlockSpec((1, D), lambda i: (0, 0)),
            pl.BlockSpec((blk, GW), lambda i: (i, 0)),
        ],
        out_specs=pl.BlockSpec((blk, D), lambda i: (i, 0)),
        out_shape=jax.ShapeDtypeStruct((n, D), jnp.float32),
    )
    return mm(x, w0, wn, b2, g)


# speedup vs baseline: 1.2054x; 1.0089x over previous
import functools

import jax
import jax.numpy as jnp
from jax import lax
from jax.experimental import pallas as pl
from jax.experimental.pallas import tpu as pltpu
from jax.experimental.pallas import tpu_sc as plsc

D = 128
K = 4
GW = K * D // 2
NC, NS = 2, 16
NW = NC * NS
CH = 64
NCH0, NCH1 = 64, 34


def _mm_body(x_ref, w0_ref, wn_ref, b_ref, g_ref, o_ref):
    g = g_ref[...]
    g_lo = lax.bitcast_convert_type(g << 16, jnp.float32)
    g_hi = lax.bitcast_convert_type(g & jnp.int32(-65536), jnp.float32)
    xg = jnp.concatenate([g_lo, g_hi], axis=1)
    o_ref[...] = (
        jnp.dot(x_ref[...], w0_ref[...], preferred_element_type=jnp.float32)
        + b_ref[...]
        + jnp.dot(xg, wn_ref[...], preferred_element_type=jnp.float32)
    )


def _make_sc_kernel(n_rows: int, np_rows: int):
    assert np_rows == NS * (NCH0 + NCH1) * CH
    assert NCH0 % 2 == 0 and NCH1 % 2 == 0
    mesh = plsc.VectorSubcoreMesh(
        core_axis_name="c", subcore_axis_name="s", num_cores=NC, num_subcores=NS
    )

    @functools.partial(
        pl.kernel,
        out_type=jax.ShapeDtypeStruct((np_rows, GW), jnp.int32),
        mesh=mesh,
        scratch_types=[
            pltpu.VMEM((max(NCH0, NCH1), K // 2, 2 * CH), jnp.int32),
            pltpu.VMEM((2, K // 2, 2 * CH, D), jnp.float32),
            pltpu.VMEM((2, CH, GW), jnp.int32),
            pltpu.SemaphoreType.DMA,
            pltpu.SemaphoreType.DMA,
            pltpu.SemaphoreType.DMA,
            pltpu.SemaphoreType.DMA,
        ],
        compiler_params=pltpu.CompilerParams(needs_layout_passes=False),
    )
    def sc_gather_pack(x_hbm, idxf_hbm, g_hbm,
                       idx_s, rows_s, gp_s, sem_g0, sem_g1, sem_o0, sem_o1):
        sid = lax.axis_index("s")
        cid = lax.axis_index("c")
        g0 = sid * (NCH0 + NCH1) + cid * NCH0
        nch_w = jnp.where(cid == 1, NCH1, NCH0)
        wbase = g0 * CH
        sem_g = (sem_g0, sem_g1)
        sem_o = (sem_o0, sem_o1)

        def in_copies(c, p):
            return [
                pltpu.make_async_copy(
                    x_hbm.at[idx_s.at[c, kk]], rows_s.at[p, kk], sem_g[p]
                )
                for kk in range(K // 2)
            ]

        def out_copy(c, p):
            base = wbase + c * CH
            return pltpu.make_async_copy(
                gp_s.at[p], g_hbm.at[pl.ds(base, CH)], sem_o[p]
            )

        def stage_and_fire(c, p):
            for cp in in_copies(c, p):
                cp.start()

        def chunk_step(c, p):
            q = 1 - p

            @pl.when(c + 1 < nch_w)
            def _fire_next():
                @pl.when(c >= 1)
                def _drain_prev_out():
                    out_copy(c - 1, q).wait()

                stage_and_fire(c + 1, q)

            for cp in in_copies(c, p):
                cp.wait()

            @plsc.parallel_loop(0, CH, unroll=4)
            def row_body(i):
                for k in range(K):
                    kk, off = k // 2, (k % 2) * CH
                    for j in range(D // 32):
                        a = rows_s[p, kk, off + i, pl.ds(j * 32, 16)]
                        bvec = rows_s[p, kk, off + i, pl.ds(j * 32 + 16, 16)]
                        packed = plsc.pack(
                            a, bvec, format=plsc.PackFormat.INTERLEAVED
                        )
                        gp_s[p, i, pl.ds(k * (D // 2) + j * 16, 16)] = (
                            plsc.bitcast(packed, jnp.int32)
                        )
            out_copy(c, p).start()

        @pl.when(cid == 0)
        def _stage_idx0():
            pltpu.sync_copy(
                idxf_hbm.at[pl.ds(g0, NCH0)], idx_s.at[pl.ds(0, NCH0)]
            )

        @pl.when(cid == 1)
        def _stage_idx1():
            pltpu.sync_copy(
                idxf_hbm.at[pl.ds(g0, NCH1)], idx_s.at[pl.ds(0, NCH1)]
            )

        stage_and_fire(0, 0)

        def pair_body(t, carry):
            chunk_step(2 * t, 0)
            chunk_step(2 * t + 1, 1)
            return carry

        lax.fori_loop(0, nch_w // 2, pair_body, 0)

        out_copy(nch_w - 2, 0).wait()
        out_copy(nch_w - 1, 1).wait()

    return sc_gather_pack


def kernel(x, neighbor_idx, W, b):
    n, d = x.shape
    assert d == D
    np_rows = ((n + NW * CH - 1) // (NW * CH)) * (NW * CH)
    blk = 8192

    idxf = (
        jnp.pad(neighbor_idx.astype(jnp.int32).T, ((0, 0), (0, np_rows - n)))
        .reshape(K, np_rows // CH, CH)
        .transpose(1, 0, 2)
        .reshape(np_rows // CH, K // 2, 2 * CH)
    )

    wt = W.T
    w0 = wt[0:D]
    pos = jnp.arange(2 * GW)
    part, w = pos // GW, pos % GW
    kk, jj, cc = w // (D // 2), (w % (D // 2)) // 16, w % 16
    orig = D * kk + 32 * jj + 16 * part + cc
    wn = wt[D:][orig]
    b2 = b.reshape(1, D)

    g = _make_sc_kernel(n, np_rows)(x, idxf)

    n_blocks = (n + blk - 1) // blk
    mm = pl.pallas_call(
        _mm_body,
        grid=(n_blocks,),
        in_specs=[
            pl.BlockSpec((blk, D), lambda i: (i, 0)),
            pl.BlockSpec((D, D), lambda i: (0, 0)),
            pl.BlockSpec((2 * GW, D), lambda i: (0, 0)),
            pl.BlockSpec((1, D), lambda i: (0, 0)),
            pl.BlockSpec((blk, GW), lambda i: (i, 0)),
        ],
        out_specs=pl.BlockSpec((blk, D), lambda i: (i, 0)),
        out_shape=jax.ShapeDtypeStruct((n, D), jnp.float32),
    )
    return mm(x, w0, wn, b2, g)


# speedup vs baseline: 1.2089x; 1.0029x over previous
import functools

import jax
import jax.numpy as jnp
from jax import lax
from jax.experimental import pallas as pl
from jax.experimental.pallas import tpu as pltpu
from jax.experimental.pallas import tpu_sc as plsc

D = 128
K = 4
GW = K * D // 2
NC, NS = 2, 16
NW = NC * NS
CH = 64
NCH0, NCH1 = 64, 34


def _mm_body(x_ref, w0_ref, wn_ref, b_ref, g_ref, o_ref):
    g = g_ref[...]
    g_lo = lax.bitcast_convert_type(g << 16, jnp.float32)
    g_hi = lax.bitcast_convert_type(g & jnp.int32(-65536), jnp.float32)
    xg = jnp.concatenate([g_lo, g_hi], axis=1)
    o_ref[...] = (
        jnp.dot(x_ref[...], w0_ref[...], preferred_element_type=jnp.float32)
        + b_ref[...]
        + jnp.dot(xg, wn_ref[...], preferred_element_type=jnp.float32)
    )


def _make_sc_kernel(n_rows: int, np_rows: int):
    assert np_rows == NS * (NCH0 + NCH1) * CH
    assert NCH0 % 2 == 0 and NCH1 % 2 == 0
    mesh = plsc.VectorSubcoreMesh(
        core_axis_name="c", subcore_axis_name="s", num_cores=NC, num_subcores=NS
    )

    @functools.partial(
        pl.kernel,
        out_type=jax.ShapeDtypeStruct((np_rows, GW), jnp.int32),
        mesh=mesh,
        scratch_types=[
            pltpu.VMEM((max(NCH0, NCH1), K // 2, 2 * CH), jnp.int32),
            pltpu.VMEM((2, K // 2, 2 * CH, D), jnp.float32),
            pltpu.VMEM((2, CH, GW), jnp.int32),
            pltpu.SemaphoreType.DMA,
            pltpu.SemaphoreType.DMA,
            pltpu.SemaphoreType.DMA,
            pltpu.SemaphoreType.DMA,
        ],
        compiler_params=pltpu.CompilerParams(needs_layout_passes=False),
    )
    def sc_gather_pack(x_hbm, idxf_hbm, g_hbm,
                       idx_s, rows_s, gp_s, sem_g0, sem_g1, sem_o0, sem_o1):
        sid = lax.axis_index("s")
        cid = lax.axis_index("c")
        g0 = sid * (NCH0 + NCH1) + cid * NCH0
        nch_w = jnp.where(cid == 1, NCH1, NCH0)
        wbase = g0 * CH
        sem_g = (sem_g0, sem_g1)
        sem_o = (sem_o0, sem_o1)

        def in_copies(c, p):
            return [
                pltpu.make_async_copy(
                    x_hbm.at[idx_s.at[c, kk]], rows_s.at[p, kk], sem_g[p]
                )
                for kk in range(K // 2)
            ]

        def out_copy(c, p):
            base = wbase + c * CH
            return pltpu.make_async_copy(
                gp_s.at[p], g_hbm.at[pl.ds(base, CH)], sem_o[p]
            )

        def stage_and_fire(c, p):
            for cp in in_copies(c, p):
                cp.start()

        def chunk_step(c, p):
            q = 1 - p

            @pl.when(c + 1 < nch_w)
            def _fire_next():
                @pl.when(c >= 1)
                def _drain_prev_out():
                    out_copy(c - 1, q).wait()

                stage_and_fire(c + 1, q)

            for cp in in_copies(c, p):
                cp.wait()

            @plsc.parallel_loop(0, CH, unroll=4)
            def row_body(i):
                for k in range(K):
                    kk, off = k // 2, (k % 2) * CH
                    for j in range(D // 32):
                        a = rows_s[p, kk, off + i, pl.ds(j * 32, 16)]
                        bvec = rows_s[p, kk, off + i, pl.ds(j * 32 + 16, 16)]
                        packed = plsc.pack(
                            a, bvec, format=plsc.PackFormat.INTERLEAVED
                        )
                        gp_s[p, i, pl.ds(k * (D // 2) + j * 16, 16)] = (
                            plsc.bitcast(packed, jnp.int32)
                        )
            out_copy(c, p).start()

        @pl.when(cid == 0)
        def _stage_idx0():
            pltpu.sync_copy(
                idxf_hbm.at[pl.ds(g0, NCH0)], idx_s.at[pl.ds(0, NCH0)]
            )

        @pl.when(cid == 1)
        def _stage_idx1():
            pltpu.sync_copy(
                idxf_hbm.at[pl.ds(g0, NCH1)], idx_s.at[pl.ds(0, NCH1)]
            )

        stage_and_fire(0, 0)

        def pair_body(t, carry):
            chunk_step(2 * t, 0)
            chunk_step(2 * t + 1, 1)
            return carry

        lax.fori_loop(0, nch_w // 2, pair_body, 0)

        out_copy(nch_w - 2, 0).wait()
        out_copy(nch_w - 1, 1).wait()

    return sc_gather_pack


def kernel(x, neighbor_idx, W, b):
    n, d = x.shape
    assert d == D
    np_rows = ((n + NW * CH - 1) // (NW * CH)) * (NW * CH)
    blk = 12544

    idxf = (
        jnp.pad(neighbor_idx.astype(jnp.int32).T, ((0, 0), (0, np_rows - n)))
        .reshape(K, np_rows // CH, CH)
        .transpose(1, 0, 2)
        .reshape(np_rows // CH, K // 2, 2 * CH)
    )

    wt = W.T
    w0 = wt[0:D]
    pos = jnp.arange(2 * GW)
    part, w = pos // GW, pos % GW
    kk, jj, cc = w // (D // 2), (w % (D // 2)) // 16, w % 16
    orig = D * kk + 32 * jj + 16 * part + cc
    wn = wt[D:][orig]
    b2 = b.reshape(1, D)

    g = _make_sc_kernel(n, np_rows)(x, idxf)

    n_blocks = (n + blk - 1) // blk
    mm = pl.pallas_call(
        _mm_body,
        grid=(n_blocks,),
        in_specs=[
            pl.BlockSpec((blk, D), lambda i: (i, 0)),
            pl.BlockSpec((D, D), lambda i: (0, 0)),
            pl.BlockSpec((2 * GW, D), lambda i: (0, 0)),
            pl.BlockSpec((1, D), lambda i: (0, 0)),
            pl.BlockSpec((blk, GW), lambda i: (i, 0)),
        ],
        out_specs=pl.BlockSpec((blk, D), lambda i: (i, 0)),
        out_shape=jax.ShapeDtypeStruct((n, D), jnp.float32),
    )
    return mm(x, w0, wn, b2, g)
